# Initial kernel scaffold; baseline (speedup 1.0000x reference)
import jax
import jax.numpy as jnp
from jax.experimental import pallas as pl


def kernel(x, edge_index, edge_attr, params):
    raise NotImplementedError("write your pallas kernel here")


# speedup vs baseline: 2.2116x; 2.2116x over previous
import functools

import jax
import jax.numpy as jnp
from jax.experimental import pallas as pl
from jax.experimental.pallas import tpu as pltpu

_D = 128
_NUM_LAYERS = 2
_EPS = 1e-16


def _dot(a, b):
    return jnp.dot(a, b, preferred_element_type=jnp.float32)


def _resid_h(xw0, b0, r1, rb1, r2, rb2):
    h = jnp.maximum(xw0 + b0, 0.0)
    r = jnp.maximum(_dot(h, r1) + rb1, 0.0)
    r = _dot(r, r2) + rb2
    return h + r


def _mlp_kernel(x_ref, w0, b0, r1, rb1, r2, rb2, wo, bo, o_ref):
    h = _resid_h(_dot(x_ref[...], w0[...]), b0[...], r1[...], rb1[...],
                 r2[...], rb2[...])
    o_ref[...] = _dot(h, wo[...]) + bo[...]


def _full_spec(a):
    return pl.BlockSpec(a.shape, lambda i: tuple(0 for _ in a.shape))


def _mlp_forward(x, p, block):
    n, ind = x.shape
    odim = p['Wo'].shape[1]
    args = (x, p['W0'], p['b0'].reshape(1, -1), p['R1'],
            p['rb1'].reshape(1, -1), p['R2'], p['rb2'].reshape(1, -1),
            p['Wo'], p['bo'].reshape(1, -1))
    in_specs = [pl.BlockSpec((block, ind), lambda i: (i, 0))]
    in_specs += [_full_spec(a) for a in args[1:]]
    return pl.pallas_call(
        _mlp_kernel,
        grid=(n // block,),
        in_specs=in_specs,
        out_specs=pl.BlockSpec((block, odim), lambda i: (i, 0)),
        out_shape=jax.ShapeDtypeStruct((n, odim), jnp.float32),
    )(*args)


def _edge_kernel(nj_ref, el_ref, w0a, w0b, b0, r1, rb1, r2, rb2, wo, bo,
                 wa, ba, scaled_ref, e16_ref):
    xw0 = _dot(nj_ref[...], w0a[...]) + _dot(el_ref[...], w0b[...])
    h = _resid_h(xw0, b0[...], r1[...], rb1[...], r2[...], rb2[...])
    nel = _dot(h, wo[...]) + bo[...]
    score = jnp.maximum(
        jnp.sum(nel * wa[...], axis=1, keepdims=True) + ba[...], 0.0)
    e = jnp.exp(score)
    scaled_ref[...] = nel * e
    e16_ref[...] = jnp.broadcast_to(e, e16_ref.shape)


def _edge_forward(node_j, edge_latents, lp, block):
    ne = node_j.shape[0]
    ep = lp['edge_proc']
    args = (node_j, edge_latents,
            ep['W0'][:_D], ep['W0'][_D:], ep['b0'].reshape(1, -1),
            ep['R1'], ep['rb1'].reshape(1, -1),
            ep['R2'], ep['rb2'].reshape(1, -1),
            ep['Wo'], ep['bo'].reshape(1, -1),
            lp['Wa'].reshape(1, -1), lp['ba'].reshape(1, 1))
    in_specs = [pl.BlockSpec((block, _D), lambda i: (i, 0)),
                pl.BlockSpec((block, _D), lambda i: (i, 0))]
    in_specs += [_full_spec(a) for a in args[2:]]
    return pl.pallas_call(
        _edge_kernel,
        grid=(ne // block,),
        in_specs=in_specs,
        out_specs=(pl.BlockSpec((block, _D), lambda i: (i, 0)),
                   pl.BlockSpec((block, 16), lambda i: (i, 0))),
        out_shape=(jax.ShapeDtypeStruct((ne, _D), jnp.float32),
                   jax.ShapeDtypeStruct((ne, 16), jnp.float32)),
    )(*args)


def _node_kernel(nl_ref, a0, a1, s0, s1, w0a, w0b, b0, r1, rb1, r2, rb2,
                 wo, bo, o_ref):
    s = s0[...][:, :1] + s1[...][:, :1]
    agg = (a0[...] + a1[...]) / (s + _EPS)
    nl = nl_ref[...]
    xw0 = _dot(nl, w0a[...]) + _dot(agg, w0b[...])
    h = _resid_h(xw0, b0[...], r1[...], rb1[...], r2[...], rb2[...])
    o_ref[...] = nl + _dot(h, wo[...]) + bo[...]


def _node_forward(node_latents, agg0, agg1, s0, s1, lp, block):
    n = node_latents.shape[0]
    np_ = lp['node_proc']
    args = (node_latents, agg0, agg1, s0, s1,
            np_['W0'][:_D], np_['W0'][_D:], np_['b0'].reshape(1, -1),
            np_['R1'], np_['rb1'].reshape(1, -1),
            np_['R2'], np_['rb2'].reshape(1, -1),
            np_['Wo'], np_['bo'].reshape(1, -1))
    in_specs = [pl.BlockSpec((block, _D), lambda i: (i, 0)),
                pl.BlockSpec((block, _D), lambda i: (i, 0)),
                pl.BlockSpec((block, _D), lambda i: (i, 0)),
                pl.BlockSpec((block, 16), lambda i: (i, 0)),
                pl.BlockSpec((block, 16), lambda i: (i, 0))]
    in_specs += [_full_spec(a) for a in args[5:]]
    return pl.pallas_call(
        _node_kernel,
        grid=(n // block,),
        in_specs=in_specs,
        out_specs=pl.BlockSpec((block, _D), lambda i: (i, 0)),
        out_shape=jax.ShapeDtypeStruct((n, _D), jnp.float32),
    )(*args)


def _forward(x, edge_index, edge_attr, params, edge_block, node_block):
    n = x.shape[0]
    src = edge_index[0]
    dst = edge_index[1]
    node_latents = _mlp_forward(x, params['node_enc'], node_block)
    edge_latents = _mlp_forward(edge_attr, params['edge_enc'], edge_block)
    zero_a = jnp.zeros((n, _D), jnp.float32)
    zero_s = jnp.zeros((n, 16), jnp.float32)
    for l in range(_NUM_LAYERS):
        lp = params['proc%d' % l]
        node_j = node_latents[src]
        scaled, e16 = _edge_forward(node_j, edge_latents, lp, edge_block)
        aggu = jax.ops.segment_sum(scaled, dst, num_segments=n)
        s16 = jax.ops.segment_sum(e16, dst, num_segments=n)
        node_latents = _node_forward(node_latents, aggu, zero_a, s16,
                                     zero_s, lp, node_block)
    dec = dict(params['decoder'])
    odim = dec['Wo'].shape[1]
    dec['Wo'] = jnp.zeros((_D, _D), jnp.float32).at[:, :odim].set(dec['Wo'])
    dec['bo'] = jnp.zeros((_D,), jnp.float32).at[:odim].set(dec['bo'])
    out = _mlp_forward(node_latents, dec, node_block)
    return out[:, :odim]


def kernel(x, edge_index, edge_attr, params):
    return _forward(x, edge_index, edge_attr, params,
                    edge_block=2000, node_block=2000)


# speedup vs baseline: 4.7495x; 2.1476x over previous
import functools

import jax
import jax.numpy as jnp
from jax import lax
from jax.experimental import pallas as pl
from jax.experimental.pallas import tpu as pltpu
from jax.experimental.pallas import tpu_sc as plsc

_D = 128
_NUM_LAYERS = 2
_EPS = 1e-16
_NC = 2
_NS = 16
_NW = _NC * _NS
_CHUNK = 80


def _dot(a, b):
    return jnp.dot(a, b, preferred_element_type=jnp.float32)


def _resid_h(xw0, b0, r1, rb1, r2, rb2):
    h = jnp.maximum(xw0 + b0, 0.0)
    r = jnp.maximum(_dot(h, r1) + rb1, 0.0)
    r = _dot(r, r2) + rb2
    return h + r


def _mlp_kernel(x_ref, w0, b0, r1, rb1, r2, rb2, wo, bo, o_ref):
    h = _resid_h(_dot(x_ref[...], w0[...]), b0[...], r1[...], rb1[...],
                 r2[...], rb2[...])
    o_ref[...] = _dot(h, wo[...]) + bo[...]


def _full_spec(a):
    return pl.BlockSpec(a.shape, lambda i: tuple(0 for _ in a.shape))


def _mlp_forward(x, p, block):
    n, ind = x.shape
    odim = p['Wo'].shape[1]
    args = (x, p['W0'], p['b0'].reshape(1, -1), p['R1'],
            p['rb1'].reshape(1, -1), p['R2'], p['rb2'].reshape(1, -1),
            p['Wo'], p['bo'].reshape(1, -1))
    in_specs = [pl.BlockSpec((block, ind), lambda i: (i, 0))]
    in_specs += [_full_spec(a) for a in args[1:]]
    return pl.pallas_call(
        _mlp_kernel,
        grid=(n // block,),
        in_specs=in_specs,
        out_specs=pl.BlockSpec((block, odim), lambda i: (i, 0)),
        out_shape=jax.ShapeDtypeStruct((n, odim), jnp.float32),
    )(*args)


def _edge_kernel(nj_ref, el_ref, w0a, w0b, b0, r1, rb1, r2, rb2, wo, bo,
                 wa, ba, scaled_ref, e16_ref):
    xw0 = _dot(nj_ref[...], w0a[...]) + _dot(el_ref[...], w0b[...])
    h = _resid_h(xw0, b0[...], r1[...], rb1[...], r2[...], rb2[...])
    nel = _dot(h, wo[...]) + bo[...]
    score = jnp.maximum(
        jnp.sum(nel * wa[...], axis=1, keepdims=True) + ba[...], 0.0)
    e = jnp.exp(score)
    scaled_ref[...] = nel * e
    e16_ref[...] = jnp.broadcast_to(e, e16_ref.shape)


def _edge_forward(node_j, edge_latents, lp, block):
    ne = node_j.shape[0]
    ep = lp['edge_proc']
    args = (node_j, edge_latents,
            ep['W0'][:_D], ep['W0'][_D:], ep['b0'].reshape(1, -1),
            ep['R1'], ep['rb1'].reshape(1, -1),
            ep['R2'], ep['rb2'].reshape(1, -1),
            ep['Wo'], ep['bo'].reshape(1, -1),
            lp['Wa'].reshape(1, -1), lp['ba'].reshape(1, 1))
    in_specs = [pl.BlockSpec((block, _D), lambda i: (i, 0)),
                pl.BlockSpec((block, _D), lambda i: (i, 0))]
    in_specs += [_full_spec(a) for a in args[2:]]
    return pl.pallas_call(
        _edge_kernel,
        grid=(ne // block,),
        in_specs=in_specs,
        out_specs=(pl.BlockSpec((block, _D), lambda i: (i, 0)),
                   pl.BlockSpec((block, 16), lambda i: (i, 0))),
        out_shape=(jax.ShapeDtypeStruct((ne, _D), jnp.float32),
                   jax.ShapeDtypeStruct((ne, 16), jnp.float32)),
    )(*args)


def _node_kernel(nl_ref, a0, a1, s0, s1, w0a, w0b, b0, r1, rb1, r2, rb2,
                 wo, bo, o_ref):
    s = s0[...][:, :1] + s1[...][:, :1]
    agg = (a0[...] + a1[...]) / (s + _EPS)
    nl = nl_ref[...]
    xw0 = _dot(nl, w0a[...]) + _dot(agg, w0b[...])
    h = _resid_h(xw0, b0[...], r1[...], rb1[...], r2[...], rb2[...])
    o_ref[...] = nl + _dot(h, wo[...]) + bo[...]


def _node_forward(node_latents, agg0, agg1, s0, s1, lp, block):
    n = node_latents.shape[0]
    np_ = lp['node_proc']
    args = (node_latents, agg0, agg1, s0, s1,
            np_['W0'][:_D], np_['W0'][_D:], np_['b0'].reshape(1, -1),
            np_['R1'], np_['rb1'].reshape(1, -1),
            np_['R2'], np_['rb2'].reshape(1, -1),
            np_['Wo'], np_['bo'].reshape(1, -1))
    in_specs = [pl.BlockSpec((block, _D), lambda i: (i, 0)),
                pl.BlockSpec((block, _D), lambda i: (i, 0)),
                pl.BlockSpec((block, _D), lambda i: (i, 0)),
                pl.BlockSpec((block, 16), lambda i: (i, 0)),
                pl.BlockSpec((block, 16), lambda i: (i, 0))]
    in_specs += [_full_spec(a) for a in args[5:]]
    return pl.pallas_call(
        _node_kernel,
        grid=(n // block,),
        in_specs=in_specs,
        out_specs=pl.BlockSpec((block, _D), lambda i: (i, 0)),
        out_shape=jax.ShapeDtypeStruct((n, _D), jnp.float32),
    )(*args)


def _sc_mesh():
    return plsc.VectorSubcoreMesh(core_axis_name="c", subcore_axis_name="s")


def _sc_gather(table, idx):
    e = idx.shape[0]
    d = table.shape[1]
    ept = e // _NW

    @functools.partial(
        pl.kernel,
        mesh=_sc_mesh(),
        out_type=jax.ShapeDtypeStruct((e, d), jnp.float32),
        scratch_types=[
            pltpu.VMEM((_CHUNK,), jnp.int32),
            pltpu.VMEM((_CHUNK, d), jnp.float32),
            pltpu.SemaphoreType.DMA,
        ],
    )
    def k(table_hbm, idx_hbm, out_hbm, idx_v, rows_v, sem):
        wid = lax.axis_index("s") * _NC + lax.axis_index("c")
        base = wid * ept

        def body(i, carry):
            off = base + i * _CHUNK
            pltpu.sync_copy(idx_hbm.at[pl.ds(off, _CHUNK)], idx_v)
            pltpu.async_copy(table_hbm.at[idx_v], rows_v, sem).wait()
            pltpu.sync_copy(rows_v, out_hbm.at[pl.ds(off, _CHUNK)])
            return carry

        lax.fori_loop(0, ept // _CHUNK, body, 0)

    return k(table, idx)


def _sc_scatter(scaled, e16, dst, n):
    e = scaled.shape[0]
    ept = e // _NW
    n_pad = ((n + 8 * _NS - 1) // (8 * _NS)) * (8 * _NS)
    rpt = n_pad // _NS
    z_a = jnp.zeros((_CHUNK, _D), jnp.float32)
    z_s = jnp.zeros((_CHUNK, 16), jnp.float32)

    @functools.partial(
        pl.kernel,
        mesh=_sc_mesh(),
        out_type=(jax.ShapeDtypeStruct((_NC * n_pad, _D), jnp.float32),
                  jax.ShapeDtypeStruct((_NC * n_pad, 16), jnp.float32)),
        scratch_types=[
            pltpu.VMEM((_CHUNK,), jnp.int32),
            pltpu.VMEM((_CHUNK,), jnp.int32),
            pltpu.VMEM((_CHUNK, _D), jnp.float32),
            pltpu.VMEM((_CHUNK, 16), jnp.float32),
            pltpu.VMEM_SHARED((n_pad, _D), jnp.float32),
            pltpu.VMEM_SHARED((n_pad, 16), jnp.float32),
        ],
    )
    def k(scaled_hbm, e16_hbm, dst_hbm, za_hbm, zs_hbm, agg_hbm, s_hbm,
          idx_v, lin_v, rows_v, erows_v, sh_a, sh_s):
        c = lax.axis_index("c")
        s = lax.axis_index("s")
        wid = s * _NC + c
        t0 = s * rpt
        lane = jnp.arange(16, dtype=jnp.int32)

        def set_lin(r0):
            for q in range(_CHUNK // 16):
                lin_v[pl.ds(q * 16, 16)] = lane + (r0 + q * 16)

        pltpu.sync_copy(za_hbm, rows_v)
        pltpu.sync_copy(zs_hbm, erows_v)

        def zbody(j, carry):
            set_lin(t0 + j * _CHUNK)
            pltpu.sync_copy(rows_v, sh_a.at[lin_v])
            pltpu.sync_copy(erows_v, sh_s.at[lin_v])
            return carry

        lax.fori_loop(0, rpt // _CHUNK, zbody, 0)
        plsc.subcore_barrier()
        base = wid * ept

        def body(i, carry):
            off = base + i * _CHUNK
            pltpu.sync_copy(dst_hbm.at[pl.ds(off, _CHUNK)], idx_v)
            pltpu.sync_copy(scaled_hbm.at[pl.ds(off, _CHUNK)], rows_v)
            pltpu.sync_copy(e16_hbm.at[pl.ds(off, _CHUNK)], erows_v)
            pltpu.sync_copy(rows_v, sh_a.at[idx_v], add=True)
            pltpu.sync_copy(erows_v, sh_s.at[idx_v], add=True)
            return carry

        lax.fori_loop(0, ept // _CHUNK, body, 0)
        plsc.subcore_barrier()
        o0 = c * n_pad + t0

        def obody(j, carry):
            set_lin(t0 + j * _CHUNK)
            w0 = o0 + j * _CHUNK
            pltpu.sync_copy(sh_a.at[lin_v], rows_v)
            pltpu.sync_copy(rows_v, agg_hbm.at[pl.ds(w0, _CHUNK)])
            pltpu.sync_copy(sh_s.at[lin_v], erows_v)
            pltpu.sync_copy(erows_v, s_hbm.at[pl.ds(w0, _CHUNK)])
            return carry

        lax.fori_loop(0, rpt // _CHUNK, obody, 0)

    agg_p, s_p = k(scaled, e16, dst, z_a, z_s)
    agg_p = agg_p.reshape(_NC, n_pad, _D)
    s_p = s_p.reshape(_NC, n_pad, 16)
    return agg_p[:, :n], s_p[:, :n]


def _forward(x, edge_index, edge_attr, params, edge_block, node_block):
    n = x.shape[0]
    src = edge_index[0]
    dst = edge_index[1]
    node_latents = _mlp_forward(x, params['node_enc'], node_block)
    edge_latents = _mlp_forward(edge_attr, params['edge_enc'], edge_block)
    for l in range(_NUM_LAYERS):
        lp = params['proc%d' % l]
        node_j = _sc_gather(node_latents, src)
        scaled, e16 = _edge_forward(node_j, edge_latents, lp, edge_block)
        agg_p, s_p = _sc_scatter(scaled, e16, dst, n)
        node_latents = _node_forward(node_latents, agg_p[0], agg_p[1],
                                     s_p[0], s_p[1], lp, node_block)
    dec = dict(params['decoder'])
    odim = dec['Wo'].shape[1]
    dec['Wo'] = jnp.zeros((_D, _D), jnp.float32).at[:, :odim].set(dec['Wo'])
    dec['bo'] = jnp.zeros((_D,), jnp.float32).at[:odim].set(dec['bo'])
    out = _mlp_forward(node_latents, dec, node_block)
    return out[:, :odim]


def kernel(x, edge_index, edge_attr, params):
    return _forward(x, edge_index, edge_attr, params,
                    edge_block=2000, node_block=2000)
